# Initial kernel scaffold; baseline (speedup 1.0000x reference)
import jax
import jax.numpy as jnp
from jax.experimental import pallas as pl


def kernel(token_ids, embed_table, W1, W2):
    raise NotImplementedError("write your pallas kernel here")


# speedup vs baseline: 1.3559x; 1.3559x over previous
import functools

import jax
import jax.numpy as jnp
from jax import lax
from jax.experimental import pallas as pl
from jax.experimental.pallas import tpu as pltpu
from jax.experimental.pallas import tpu_sc as plsc

D_SYM = 64
D_MODEL = 128

_NC = 2
_NS = 16
_NW = _NC * _NS

_G = 128
_GROUP = 8
_ROWS = _G * _GROUP


def _gather_sc(table, ids2d, n_rows):
    per_w = n_rows // _NW
    n_g = per_w // _G
    n_groups = per_w // _ROWS
    mesh = plsc.VectorSubcoreMesh(core_axis_name="c", subcore_axis_name="s")

    @functools.partial(
        pl.kernel,
        mesh=mesh,
        compiler_params=pltpu.CompilerParams(use_tc_tiling_on_sc=False),
        out_type=jax.ShapeDtypeStruct((n_rows, D_SYM), jnp.float32),
        scratch_types=[
            pltpu.VMEM((n_g, _G), jnp.int32),
            pltpu.VMEM((_ROWS, D_SYM), jnp.float32),
            pltpu.SemaphoreType.DMA,
        ],
    )
    def gather_kernel(ids_hbm, table_hbm, out_hbm, idx_v, rows_v, sem):
        wid = lax.axis_index("s") * _NC + lax.axis_index("c")
        base = wid * per_w
        pltpu.sync_copy(ids_hbm.at[pl.ds(wid * n_g, n_g)], idx_v)

        def group_body(g, _):
            copies = []
            for j in range(_GROUP):
                cp = pltpu.make_async_copy(
                    table_hbm.at[idx_v.at[g * _GROUP + j]],
                    rows_v.at[pl.ds(j * _G, _G)],
                    sem,
                )
                cp.start()
                copies.append(cp)
            for cp in copies:
                cp.wait()
            pltpu.sync_copy(rows_v, out_hbm.at[pl.ds(base + g * _ROWS, _ROWS)])
            return 0

        lax.fori_loop(0, n_groups, group_body, 0)

    return gather_kernel(ids2d, table)


def _mlp_body(x_ref, w1_ref, w2_ref, o_ref):
    h = jnp.dot(x_ref[...], w1_ref[...], preferred_element_type=jnp.float32)
    h = 0.5 * h * (1.0 + lax.erf(h * 0.7071067811865476))
    o_ref[...] = jnp.dot(h, w2_ref[...], preferred_element_type=jnp.float32)


def _mlp_tc(sym, W1, W2, blk=2048):
    n = sym.shape[0]
    return pl.pallas_call(
        _mlp_body,
        grid=(n // blk,),
        in_specs=[
            pl.BlockSpec((blk, D_SYM), lambda i: (i, 0)),
            pl.BlockSpec((D_SYM, D_MODEL), lambda i: (0, 0)),
            pl.BlockSpec((D_MODEL, D_MODEL), lambda i: (0, 0)),
        ],
        out_specs=pl.BlockSpec((blk, D_MODEL), lambda i: (i, 0)),
        out_shape=jax.ShapeDtypeStruct((n, D_MODEL), jnp.float32),
    )(sym, W1, W2)


def kernel(token_ids, embed_table, W1, W2):
    B, L = token_ids.shape
    n = B * L
    ids2d = token_ids.reshape(n // _G, _G).astype(jnp.int32)
    sym = _gather_sc(embed_table, ids2d, n)
    out = _mlp_tc(sym, W1, W2)
    return out.reshape(B, L, D_MODEL)
